# baseline (speedup 1.0000x reference)
import functools

import jax
import jax.numpy as jnp
from jax import lax
from jax.experimental import pallas as pl
from jax.experimental.pallas import tpu as pltpu
from jax.experimental.pallas import tpu_sc as plsc

_B, _S, _D, _E = 4, 4096, 2048, 16
_N = _B * _S
_BT = 1024
_NCHUNK = 4
_CN = _N // _NCHUNK
_NC, _NS = 2, 16
_NW = _NC * _NS
_TPW = _CN // _NW
_L = 16
_GRP = _TPW // _L


def _logits_body(h_ref, w_ref, b_ref, out_ref):
    acc = lax.dot_general(
        h_ref[...], w_ref[...],
        (((1,), (0,)), ((), ())),
        preferred_element_type=jnp.float32,
    )
    out_ref[...] = acc + b_ref[...]


def _logits_tc(h, wt, b2, chunk):
    nblk = _CN // _BT
    return pl.pallas_call(
        _logits_body,
        grid=(nblk,),
        in_specs=[
            pl.BlockSpec((_BT, _D), lambda i, c=chunk: (i + c * nblk, 0)),
            pl.BlockSpec((_D, _E), lambda i: (0, 0)),
            pl.BlockSpec((1, _E), lambda i: (0, 0)),
        ],
        out_specs=pl.BlockSpec((_BT, _E), lambda i: (i, 0)),
        out_shape=jax.ShapeDtypeStruct((_CN, _E), jnp.float32),
    )(h, wt, b2)


def _make_router_sc(chunk):
    @functools.partial(
        pl.kernel,
        mesh=plsc.VectorSubcoreMesh(core_axis_name="c", subcore_axis_name="s"),
        out_type=(
            jax.ShapeDtypeStruct((_E, _N), jnp.float32),
            jax.ShapeDtypeStruct((_CN, 2), jnp.int32),
        ),
        scratch_types=[
            pltpu.VMEM((_TPW, _E), jnp.float32),
            pltpu.VMEM((_E, _TPW), jnp.float32),
            pltpu.VMEM((_TPW, 2), jnp.int32),
            pltpu.VMEM((_GRP, _L), jnp.float32),
            pltpu.VMEM((_L, _E), jnp.float32),
        ],
        compiler_params=pltpu.CompilerParams(
            needs_layout_passes=False, use_tc_tiling_on_sc=True
        ),
    )
    def _router_sc(logits_hbm, mmf_hbm, vb_hbm, scores_hbm, idx_hbm,
                   lblk, sblk, iblk, mblk, vblk):
        wid = lax.axis_index("s") * _NC + lax.axis_index("c")
        base = wid * _TPW
        pltpu.sync_copy(logits_hbm.at[pl.ds(base, _TPW), :], lblk)
        pltpu.sync_copy(mmf_hbm.at[pl.ds(wid * _GRP, _GRP), :], mblk)
        pltpu.sync_copy(vb_hbm, vblk)
        lanes = lax.iota(jnp.int32, _L)
        vbs = [plsc.load_gather(vblk, [lanes, jnp.full((_L,), e, jnp.int32)])
               for e in range(_E)]

        def group(g, carry):
            row = g * _L + lanes
            mask = plsc.load_gather(mblk, [jnp.full((_L,), g, jnp.int32), lanes])
            v0 = jnp.full((_L,), -jnp.inf, jnp.float32)
            v1 = jnp.full((_L,), -jnp.inf, jnp.float32)
            i0 = jnp.zeros((_L,), jnp.int32)
            i1 = jnp.zeros((_L,), jnp.int32)
            for e in range(_E):
                ev = jnp.full((_L,), e, jnp.int32)
                col = plsc.load_gather(lblk, [row, ev]) + mask * vbs[e]
                gt0 = col > v0
                gt1 = col > v1
                v1 = jnp.where(gt0, v0, jnp.where(gt1, col, v1))
                i1 = jnp.where(gt0, i0, jnp.where(gt1, ev, i1))
                v0 = jnp.where(gt0, col, v0)
                i0 = jnp.where(gt0, ev, i0)
            t = jnp.exp(v1 - v0)
            denom = 1.0 + t
            p0 = 1.0 / denom
            p1 = t / denom
            zero = jnp.zeros((_L,), jnp.float32)
            for e in range(_E):
                ev = jnp.full((_L,), e, jnp.int32)
                se = (jnp.where(i0 == ev, p0, zero)
                      + jnp.where(i1 == ev, p1, zero))
                sblk[e, pl.ds(g * _L, _L)] = se
            plsc.store_scatter(iblk, [row, jnp.zeros((_L,), jnp.int32)], i0)
            plsc.store_scatter(iblk, [row, jnp.ones((_L,), jnp.int32)], i1)
            return carry

        lax.fori_loop(0, _GRP, group, 0)
        pltpu.sync_copy(
            sblk, scores_hbm.at[:, pl.ds(chunk * _CN + base, _TPW)])
        pltpu.sync_copy(iblk, idx_hbm.at[pl.ds(base, _TPW), :])

    return _router_sc


_router_sc_chunks = [_make_router_sc(c) for c in range(_NCHUNK)]


def kernel(hidden_states, modality_mask, weight, bias, vision_bias):
    h = hidden_states.reshape(_N, _D)
    mmf = modality_mask.reshape(_N).astype(jnp.float32)
    b2 = bias.reshape(1, _E)
    wt = weight.T
    vb16 = jnp.broadcast_to(vision_bias[None, :], (_L, _E))
    scores_parts = []
    idx_parts = []
    for c in range(_NCHUNK):
        logits_c = _logits_tc(h, wt, b2, c)
        mmf_c = lax.dynamic_slice(mmf, (c * _CN,), (_CN,)).reshape(_NW * _GRP, _L)
        st_c, i_c = _router_sc_chunks[c](logits_c, mmf_c, vb16)
        scores_parts.append(st_c)
        idx_parts.append(i_c)
    col = lax.broadcasted_iota(jnp.int32, (_E, _N), 1)
    parts = scores_parts
    bound = _CN
    while len(parts) > 1:
        parts = [jnp.where((col // bound) % 2 == 0, parts[i], parts[i + 1])
                 for i in range(0, len(parts), 2)]
        bound *= 2
    scores = parts[0].T
    indices = jnp.concatenate(idx_parts, axis=0)
    return scores, indices


# speedup vs baseline: 1.1878x; 1.1878x over previous
import functools

import jax
import jax.numpy as jnp
from jax import lax
from jax.experimental import pallas as pl
from jax.experimental.pallas import tpu as pltpu
from jax.experimental.pallas import tpu_sc as plsc

_B, _S, _D, _E = 4, 4096, 2048, 16
_N = _B * _S
_BT = 1024
_NCHUNK = 2
_CN = _N // _NCHUNK
_NC, _NS = 2, 16
_NW = _NC * _NS
_TPW = _CN // _NW
_L = 16
_GRP = _TPW // _L


def _logits_body(h_ref, w_ref, b_ref, out_ref):
    acc = lax.dot_general(
        h_ref[...], w_ref[...],
        (((1,), (1,)), ((), ())),
        preferred_element_type=jnp.float32,
    )
    out_ref[...] = acc + b_ref[...]


def _logits_tc(h, wt, b2, chunk):
    nblk = _CN // _BT
    return pl.pallas_call(
        _logits_body,
        grid=(nblk,),
        in_specs=[
            pl.BlockSpec((_BT, _D), lambda i, c=chunk: (i + c * nblk, 0)),
            pl.BlockSpec((_E, _D), lambda i: (0, 0)),
            pl.BlockSpec((1, _E), lambda i: (0, 0)),
        ],
        out_specs=pl.BlockSpec((_BT, _E), lambda i: (i, 0)),
        out_shape=jax.ShapeDtypeStruct((_CN, _E), jnp.float32),
    )(h, wt, b2)


def _make_router_sc(chunk):
    @functools.partial(
        pl.kernel,
        mesh=plsc.VectorSubcoreMesh(core_axis_name="c", subcore_axis_name="s"),
        out_type=(
            jax.ShapeDtypeStruct((_E, _N), jnp.float32),
            jax.ShapeDtypeStruct((_CN, 2), jnp.int32),
        ),
        scratch_types=[
            pltpu.VMEM((_TPW, _E), jnp.float32),
            pltpu.VMEM((_E, _TPW), jnp.float32),
            pltpu.VMEM((_TPW, 2), jnp.int32),
            pltpu.VMEM((_GRP, _L), jnp.float32),
            pltpu.VMEM((_L, _E), jnp.float32),
        ],
        compiler_params=pltpu.CompilerParams(
            needs_layout_passes=False, use_tc_tiling_on_sc=True
        ),
    )
    def _router_sc(logits_hbm, mmf_hbm, vb_hbm, scores_hbm, idx_hbm,
                   lblk, sblk, iblk, mblk, vblk):
        wid = lax.axis_index("s") * _NC + lax.axis_index("c")
        base = wid * _TPW
        pltpu.sync_copy(logits_hbm.at[pl.ds(base, _TPW), :], lblk)
        pltpu.sync_copy(mmf_hbm.at[pl.ds(wid * _GRP, _GRP), :], mblk)
        pltpu.sync_copy(vb_hbm, vblk)
        lanes = lax.iota(jnp.int32, _L)
        vbs = [plsc.load_gather(vblk, [lanes, jnp.full((_L,), e, jnp.int32)])
               for e in range(_E)]

        def group(g, carry):
            row = g * _L + lanes
            mask = plsc.load_gather(mblk, [jnp.full((_L,), g, jnp.int32), lanes])
            v0 = jnp.full((_L,), -jnp.inf, jnp.float32)
            v1 = jnp.full((_L,), -jnp.inf, jnp.float32)
            i0 = jnp.zeros((_L,), jnp.int32)
            i1 = jnp.zeros((_L,), jnp.int32)
            for e in range(_E):
                ev = jnp.full((_L,), e, jnp.int32)
                col = plsc.load_gather(lblk, [row, ev]) + mask * vbs[e]
                gt0 = col > v0
                gt1 = col > v1
                v1 = jnp.where(gt0, v0, jnp.where(gt1, col, v1))
                i1 = jnp.where(gt0, i0, jnp.where(gt1, ev, i1))
                v0 = jnp.where(gt0, col, v0)
                i0 = jnp.where(gt0, ev, i0)
            t = jnp.exp(v1 - v0)
            denom = 1.0 + t
            p0 = 1.0 / denom
            p1 = t / denom
            zero = jnp.zeros((_L,), jnp.float32)
            for e in range(_E):
                ev = jnp.full((_L,), e, jnp.int32)
                se = (jnp.where(i0 == ev, p0, zero)
                      + jnp.where(i1 == ev, p1, zero))
                sblk[e, pl.ds(g * _L, _L)] = se
            plsc.store_scatter(iblk, [row, jnp.zeros((_L,), jnp.int32)], i0)
            plsc.store_scatter(iblk, [row, jnp.ones((_L,), jnp.int32)], i1)
            return carry

        lax.fori_loop(0, _GRP, group, 0)
        pltpu.sync_copy(
            sblk, scores_hbm.at[:, pl.ds(chunk * _CN + base, _TPW)])
        pltpu.sync_copy(iblk, idx_hbm.at[pl.ds(base, _TPW), :])

    return _router_sc


_router_sc_chunks = [_make_router_sc(c) for c in range(_NCHUNK)]


def kernel(hidden_states, modality_mask, weight, bias, vision_bias):
    h = hidden_states.reshape(_N, _D)
    mmf = modality_mask.reshape(_N).astype(jnp.float32)
    b2 = bias.reshape(1, _E)
    wt = weight
    vb16 = jnp.broadcast_to(vision_bias[None, :], (_L, _E))
    scores_parts = []
    idx_parts = []
    for c in range(_NCHUNK):
        logits_c = _logits_tc(h, wt, b2, c)
        mmf_c = lax.dynamic_slice(mmf, (c * _CN,), (_CN,)).reshape(_NW * _GRP, _L)
        st_c, i_c = _router_sc_chunks[c](logits_c, mmf_c, vb16)
        scores_parts.append(st_c)
        idx_parts.append(i_c)
    col = lax.broadcasted_iota(jnp.int32, (_E, _N), 1)
    parts = scores_parts
    bound = _CN
    while len(parts) > 1:
        parts = [jnp.where((col // bound) % 2 == 0, parts[i], parts[i + 1])
                 for i in range(0, len(parts), 2)]
        bound *= 2
    scores = parts[0].T
    indices = jnp.concatenate(idx_parts, axis=0)
    return scores, indices
